# baseline (speedup 1.0000x reference)
import functools

import jax
import jax.numpy as jnp
from jax import lax
from jax.experimental import pallas as pl
from jax.experimental.pallas import tpu as pltpu
from jax.experimental.pallas import tpu_sc as plsc

DIM = 32
NUM_CORES = 2
NUM_SUBCORES = 16
NW = NUM_CORES * NUM_SUBCORES

CHUNK = 800
NBUF = 4


@functools.lru_cache(maxsize=None)
def _make_gather(n_rows: int):
    assert n_rows % NW == 0
    b_per_w = n_rows // NW
    assert b_per_w % (CHUNK * NBUF) == 0
    n_chunks = b_per_w // CHUNK

    mesh = plsc.VectorSubcoreMesh(core_axis_name="c", subcore_axis_name="s")

    scratch = (
        [pltpu.VMEM((CHUNK,), jnp.int32) for _ in range(NBUF)]
        + [pltpu.VMEM((CHUNK, DIM), jnp.float32) for _ in range(NBUF)]
        + [pltpu.SemaphoreType.DMA for _ in range(3 * NBUF)]
    )

    @functools.partial(
        pl.kernel,
        mesh=mesh,
        compiler_params=pltpu.CompilerParams(use_tc_tiling_on_sc=False),
        out_type=jax.ShapeDtypeStruct((n_rows, DIM), jnp.float32),
        scratch_types=scratch,
    )
    def gather(idx_hbm, table_hbm, out_hbm, *bufs):
        idx_v = bufs[:NBUF]
        rows_v = bufs[NBUF:2 * NBUF]
        isem = bufs[2 * NBUF:3 * NBUF]
        gsem = bufs[3 * NBUF:4 * NBUF]
        wsem = bufs[4 * NBUF:5 * NBUF]

        wid = lax.axis_index("s") * NUM_CORES + lax.axis_index("c")
        base = wid * b_per_w

        def idx_copy(g, b):
            return pltpu.make_async_copy(
                idx_hbm.at[pl.ds(base + g * CHUNK, CHUNK)], idx_v[b], isem[b])

        def gather_copy(b):
            return pltpu.make_async_copy(table_hbm.at[idx_v[b]], rows_v[b], gsem[b])

        def write_copy(g, b):
            return pltpu.make_async_copy(
                rows_v[b], out_hbm.at[pl.ds(base + g * CHUNK, CHUNK)], wsem[b])

        for b in range(NBUF):
            idx_copy(b, b).start()
        for b in range(NBUF):
            idx_copy(b, b).wait()
            gather_copy(b).start()

        def body(t, carry):
            g0 = t * NBUF
            for b in range(NBUF):
                g = g0 + b
                gather_copy(b).wait()
                write_copy(g, b).start()

                @pl.when(g + NBUF < n_chunks)
                def _():
                    idx_copy(g + NBUF, b).start()

            for b in range(NBUF):
                g = g0 + b

                @pl.when(g + NBUF < n_chunks)
                def _():
                    idx_copy(g + NBUF, b).wait()
                    write_copy(g, b).wait()
                    gather_copy(b).start()

            return carry

        lax.fori_loop(0, n_chunks // NBUF, body, 0)

        for b in range(NBUF):
            write_copy(n_chunks - NBUF + b, b).wait()

    return gather


def kernel(indices, table):
    batch, hist = indices.shape
    flat_idx = indices.reshape(-1).astype(jnp.int32)
    out = _make_gather(batch * hist)(flat_idx, table)
    return out.reshape(batch, hist, DIM)


# speedup vs baseline: 1.0038x; 1.0038x over previous
import functools

import jax
import jax.numpy as jnp
from jax import lax
from jax.experimental import pallas as pl
from jax.experimental.pallas import tpu as pltpu
from jax.experimental.pallas import tpu_sc as plsc

NUM_CORES = 2
NUM_SUBCORES = 16
NW = NUM_CORES * NUM_SUBCORES

BATCH = 16384
HIST = 200
DIM = 32
H8 = HIST // 8
B128 = BATCH // 128
UNITS = H8 * B128
UPW = UNITS // NW
UNROLL = 8


def _transpose_half(rows_v, tv, half):
    lanes = lax.iota(jnp.int32, 16)
    d4v0 = lanes // 8
    d4v1 = d4v0 + 2
    d8off = (lanes % 8) * 128

    def body(i, carry):
        for dj in range(UNROLL):
            j = i * UNROLL + dj
            row = half * 512 + j
            hb = jnp.full((16,), j >> 7, jnp.int32)
            db = d8off + jnp.full((16,), j & 127, jnp.int32)
            v0 = rows_v[row, pl.ds(0, 16)]
            v1 = rows_v[row, pl.ds(16, 16)]
            plsc.store_scatter(tv, [hb, d4v0, db], v0)
            plsc.store_scatter(tv, [hb, d4v1, db], v1)
        return carry

    lax.fori_loop(0, 512 // UNROLL, body, 0)


@functools.lru_cache(maxsize=None)
def _make_gather():
    mesh = plsc.VectorSubcoreMesh(core_axis_name="c", subcore_axis_name="s")

    @functools.partial(
        pl.kernel,
        mesh=mesh,
        compiler_params=pltpu.CompilerParams(use_tc_tiling_on_sc=False, needs_layout_passes=False),
        out_type=jax.ShapeDtypeStruct((HIST, 4, 128, 1024), jnp.float32),
        scratch_types=[
            pltpu.VMEM((1024,), jnp.int32),
            pltpu.VMEM((1024,), jnp.int32),
            pltpu.VMEM((1024, DIM), jnp.float32),
            pltpu.VMEM((1024, DIM), jnp.float32),
            pltpu.VMEM((4, 4, 1024), jnp.float32),
            pltpu.VMEM((4, 4, 1024), jnp.float32),
        ] + [pltpu.SemaphoreType.DMA] * 6,
    )
    def gather(idx_hbm, table_hbm, out_hbm, iv0, iv1, rv0, rv1, tv0, tv1,
               is0, is1, gs0, gs1, ws0, ws1):
        iv = (iv0, iv1)
        rv = (rv0, rv1)
        tv = (tv0, tv1)
        isem = (is0, is1)
        gsem = (gs0, gs1)
        wsem = (ws0, ws1)

        wid = lax.axis_index("s") * NUM_CORES + lax.axis_index("c")
        base = wid * UPW

        def idx_cp(u, s):
            g = base + u
            return pltpu.make_async_copy(
                idx_hbm.at[g // B128, g % B128], iv[s], isem[s])

        def gath_cp(s):
            return pltpu.make_async_copy(
                table_hbm.at[iv[s]], rv[s], gsem[s])

        def wr_cp(u, half):
            g = base + u
            dst = out_hbm.at[pl.ds((g // B128) * 8 + half * 4, 4), :, g % B128]
            return pltpu.make_async_copy(tv[half], dst, wsem[half])

        idx_cp(0, 0).start()
        idx_cp(1, 1).start()
        idx_cp(0, 0).wait()
        gath_cp(0).start()
        idx_cp(1, 1).wait()
        gath_cp(1).start()

        def body(t, carry):
            for p in range(2):
                u = t * 2 + p
                gath_cp(p).wait()

                @pl.when(u + 2 < UPW)
                def _():
                    idx_cp(u + 2, p).start()

                for half in range(2):
                    @pl.when(u >= 1)
                    def _():
                        wr_cp(u - 1, half).wait()

                    _transpose_half(rv[p], tv[half], half)
                    wr_cp(u, half).start()

                @pl.when(u + 2 < UPW)
                def _():
                    idx_cp(u + 2, p).wait()
                    gath_cp(p).start()

            return carry

        lax.fori_loop(0, UPW // 2, body, 0)

        wr_cp(UPW - 1, 0).wait()
        wr_cp(UPW - 1, 1).wait()

    return gather


def kernel(indices, table):
    idx3 = (indices.astype(jnp.int32)
            .reshape(128, 128, 25, 8)
            .transpose(2, 0, 3, 1)
            .reshape(25, 128, 1024))
    out5 = _make_gather()(idx3, table)
    return (out5.reshape(HIST, 4, 128, 8, 128)
            .transpose(2, 4, 0, 1, 3)
            .reshape(BATCH, HIST, DIM))


# speedup vs baseline: 2.9469x; 2.9357x over previous
import functools

import jax
import jax.numpy as jnp
from jax import lax
from jax.experimental import pallas as pl
from jax.experimental.pallas import tpu as pltpu
from jax.experimental.pallas import tpu_sc as plsc

NUM_CORES = 2
NUM_SUBCORES = 16
NW = NUM_CORES * NUM_SUBCORES

BATCH = 16384
HIST = 200
DIM = 32
H8 = HIST // 8
B128 = BATCH // 128
UNITS = H8 * B128
UPW = UNITS // NW
UNROLL = 8


def _transpose_half(rows_v, tv, half):
    lanes = lax.iota(jnp.int32, 16)
    d4v0 = lanes >> 3
    d4v1 = d4v0 + 2
    d8v = lanes & 7

    for hl in range(4):
        hb = jnp.full((16,), hl, jnp.int32)
        base_row = half * 512 + hl * 128

        @plsc.parallel_loop(0, 128, step=UNROLL)
        def _(bl0):
            for dbl in range(UNROLL):
                bl = bl0 + dbl
                db = jnp.full((16,), bl, jnp.int32)
                v0 = rows_v[base_row + bl, pl.ds(0, 16)]
                v1 = rows_v[base_row + bl, pl.ds(16, 16)]
                plsc.store_scatter(tv, [hb, d4v0, d8v, db], v0)
                plsc.store_scatter(tv, [hb, d4v1, d8v, db], v1)


@functools.lru_cache(maxsize=None)
def _make_gather():
    mesh = plsc.VectorSubcoreMesh(core_axis_name="c", subcore_axis_name="s")

    @functools.partial(
        pl.kernel,
        mesh=mesh,
        compiler_params=pltpu.CompilerParams(use_tc_tiling_on_sc=False, needs_layout_passes=False),
        out_type=jax.ShapeDtypeStruct((HIST, 4, 128, 8, 128), jnp.float32),
        scratch_types=[
            pltpu.VMEM((1024,), jnp.int32),
            pltpu.VMEM((1024,), jnp.int32),
            pltpu.VMEM((1024, DIM), jnp.float32),
            pltpu.VMEM((1024, DIM), jnp.float32),
            pltpu.VMEM((4, 4, 8, 129), jnp.float32),
            pltpu.VMEM((4, 4, 8, 129), jnp.float32),
        ] + [pltpu.SemaphoreType.DMA] * 6,
    )
    def gather(idx_hbm, table_hbm, out_hbm, iv0, iv1, rv0, rv1, tv0, tv1,
               is0, is1, gs0, gs1, ws0, ws1):
        iv = (iv0, iv1)
        rv = (rv0, rv1)
        tv = (tv0, tv1)
        isem = (is0, is1)
        gsem = (gs0, gs1)
        wsem = (ws0, ws1)

        wid = lax.axis_index("s") * NUM_CORES + lax.axis_index("c")
        base = wid * UPW

        def idx_cp(u, s):
            g = base + u
            return pltpu.make_async_copy(
                idx_hbm.at[g // B128, g % B128], iv[s], isem[s])

        def gath_cp(s):
            return pltpu.make_async_copy(
                table_hbm.at[iv[s]], rv[s], gsem[s])

        def wr_cp(u, half):
            g = base + u
            dst = out_hbm.at[pl.ds((g // B128) * 8 + half * 4, 4), :, g % B128]
            src = tv[half].at[:, :, :, pl.ds(0, 128)]
            return pltpu.make_async_copy(src, dst, wsem[half])

        idx_cp(0, 0).start()
        idx_cp(1, 1).start()
        idx_cp(0, 0).wait()
        gath_cp(0).start()
        idx_cp(1, 1).wait()
        gath_cp(1).start()

        def body(t, carry):
            for p in range(2):
                u = t * 2 + p
                gath_cp(p).wait()

                @pl.when(u + 2 < UPW)
                def _():
                    idx_cp(u + 2, p).start()

                for half in range(2):
                    @pl.when(u >= 1)
                    def _():
                        wr_cp(u - 1, half).wait()

                    _transpose_half(rv[p], tv[half], half)
                    wr_cp(u, half).start()

                @pl.when(u + 2 < UPW)
                def _():
                    idx_cp(u + 2, p).wait()
                    gath_cp(p).start()

            return carry

        lax.fori_loop(0, UPW // 2, body, 0)

        wr_cp(UPW - 1, 0).wait()
        wr_cp(UPW - 1, 1).wait()

    return gather


def kernel(indices, table):
    idx3 = (indices.astype(jnp.int32)
            .reshape(128, 128, 25, 8)
            .transpose(2, 0, 3, 1)
            .reshape(25, 128, 1024))
    out5 = _make_gather()(idx3, table)
    return (out5.transpose(2, 4, 0, 1, 3)
            .reshape(BATCH, HIST, DIM))
